# Initial kernel scaffold; baseline (speedup 1.0000x reference)
import jax
import jax.numpy as jnp
from jax.experimental import pallas as pl


def kernel(x, edge_index, edge_attr, We1, be1, We2, be2, We3, be3, lne_g, lne_b, Wn1, bn1, Wn2, bn2, Wn3, bn3, lnn_g, lnn_b):
    raise NotImplementedError("write your pallas kernel here")


# speedup vs baseline: 2.8491x; 2.8491x over previous
import functools

import jax
import jax.numpy as jnp
from jax import lax
from jax.experimental import pallas as pl
from jax.experimental.pallas import tpu as pltpu
from jax.experimental.pallas import tpu_sc as plsc

NC = 2
NS = 16
NW = NC * NS

C = 80
ROWS_PER_TILE = 640
N_PAD = NS * ROWS_PER_TILE


def _gather_combine(E, N, D):
    per_w = E // NW
    n_chunks = per_w // C
    assert per_w % C == 0 and E % NW == 0
    mesh = plsc.VectorSubcoreMesh(core_axis_name="c", subcore_axis_name="s")

    @functools.partial(
        pl.kernel,
        mesh=mesh,
        out_type=jax.ShapeDtypeStruct((E, D), jnp.float32),
        scratch_types=[
            pltpu.VMEM((C,), jnp.int32),
            pltpu.VMEM((C,), jnp.int32),
            pltpu.VMEM((C, D), jnp.float32),
            pltpu.VMEM((C, D), jnp.float32),
            pltpu.SemaphoreType.DMA,
            pltpu.SemaphoreType.DMA,
        ],
    )
    def k(pd_hbm, ps_hbm, dst_hbm, src_hbm, out_hbm, idxd, idxs, bufa, bufb,
          sem1, sem2):
        wid = lax.axis_index("s") * NC + lax.axis_index("c")
        base = wid * per_w

        def chunk(i, carry):
            off = base + i * C
            pltpu.sync_copy(dst_hbm.at[pl.ds(off, C)], idxd)
            pltpu.sync_copy(src_hbm.at[pl.ds(off, C)], idxs)
            cp1 = pltpu.async_copy(pd_hbm.at[idxd], bufa, sem1)
            cp2 = pltpu.async_copy(ps_hbm.at[idxs], bufb, sem2)
            cp1.wait()
            cp2.wait()

            def row(r, c2):
                for j in range(D // 16):
                    sl = pl.ds(j * 16, 16)
                    bufa[r, sl] = bufa[r, sl] + bufb[r, sl]
                return c2

            lax.fori_loop(0, C, row, 0)
            pltpu.sync_copy(bufa, out_hbm.at[pl.ds(off, C)])
            return carry

        lax.fori_loop(0, n_chunks, chunk, 0)

    return k


def _scatter_partial(E, D):
    per_t = E // NW
    n_chunks = per_t // C
    nz_chunks = ROWS_PER_TILE // C
    mesh = plsc.VectorSubcoreMesh(core_axis_name="c", subcore_axis_name="s")

    @functools.partial(
        pl.kernel,
        mesh=mesh,
        out_type=jax.ShapeDtypeStruct((NC, N_PAD, D), jnp.float32),
        scratch_types=[
            pltpu.VMEM((C,), jnp.int32),
            pltpu.VMEM((C, D), jnp.float32),
            pltpu.VMEM((C, D), jnp.float32),
            pltpu.VMEM_SHARED((N_PAD, D), jnp.float32),
        ],
    )
    def k(h_hbm, dst_hbm, out_hbm, idxd, bufh, zbuf, acc):
        c = lax.axis_index("c")
        s = lax.axis_index("s")

        def zrow(r, carry):
            for j in range(D // 16):
                zbuf[r, pl.ds(j * 16, 16)] = jnp.zeros((16,), jnp.float32)
            return carry

        lax.fori_loop(0, C, zrow, 0)
        row0 = s * ROWS_PER_TILE

        def zchunk(i, carry):
            pltpu.sync_copy(zbuf, acc.at[pl.ds(row0 + i * C, C)])
            return carry

        lax.fori_loop(0, nz_chunks, zchunk, 0)
        plsc.subcore_barrier()

        base = c * (E // NC) + s * per_t

        def chunk(i, carry):
            off = base + i * C
            pltpu.sync_copy(dst_hbm.at[pl.ds(off, C)], idxd)
            pltpu.sync_copy(h_hbm.at[pl.ds(off, C)], bufh)
            pltpu.sync_copy(bufh, acc.at[idxd], add=True)
            return carry

        lax.fori_loop(0, n_chunks, chunk, 0)
        plsc.subcore_barrier()

        def ochunk(i, carry):
            r0 = row0 + i * C
            pltpu.sync_copy(acc.at[pl.ds(r0, C)], out_hbm.at[c, pl.ds(r0, C)])
            return carry

        lax.fori_loop(0, nz_chunks, ochunk, 0)

    return k


def _edge_mlp(E, D, H, BE):
    grid = (E // BE,)

    def body(ea, g, wa, w2, w3, b1, b2, b3, lg, lb, out_ea, out_h):
        h1 = jnp.maximum(
            jnp.dot(ea[...], wa[...], preferred_element_type=jnp.float32)
            + g[...] + b1[...], 0.0)
        h2 = jnp.maximum(
            jnp.dot(h1, w2[...], preferred_element_type=jnp.float32)
            + b2[...], 0.0)
        h3 = jnp.dot(h2, w3[...], preferred_element_type=jnp.float32) + b3[...]
        m = jnp.mean(h3, axis=-1, keepdims=True)
        v = jnp.mean((h3 - m) ** 2, axis=-1, keepdims=True)
        h = (h3 - m) * lax.rsqrt(v + 1e-5) * lg[...] + lb[...]
        out_ea[...] = ea[...] + h
        out_h[...] = h

    ebs = pl.BlockSpec((BE, D), lambda i: (i, 0))
    wbs = pl.BlockSpec((D, H), lambda i: (0, 0))
    bbs = pl.BlockSpec((1, H), lambda i: (0, 0))
    return pl.pallas_call(
        body,
        grid=grid,
        in_specs=[ebs, ebs, wbs, wbs, wbs, bbs, bbs, bbs, bbs, bbs],
        out_specs=[ebs, ebs],
        out_shape=[
            jax.ShapeDtypeStruct((E, D), jnp.float32),
            jax.ShapeDtypeStruct((E, D), jnp.float32),
        ],
    )


def _node_mlp(N, D, H, BN, emit_next):
    grid = (N // BN,)

    def body(*refs):
        if emit_next:
            (x, a0, a1, w1x, w1a, w2, w3, b1, b2, b3, lg, lb, wd, ws,
             out_x, out_pd, out_ps) = refs
        else:
            (x, a0, a1, w1x, w1a, w2, w3, b1, b2, b3, lg, lb,
             out_x) = refs
        aggr = a0[0] + a1[0]
        g1 = jnp.maximum(
            jnp.dot(x[...], w1x[...], preferred_element_type=jnp.float32)
            + jnp.dot(aggr, w1a[...], preferred_element_type=jnp.float32)
            + b1[...], 0.0)
        g2 = jnp.maximum(
            jnp.dot(g1, w2[...], preferred_element_type=jnp.float32)
            + b2[...], 0.0)
        g3 = jnp.dot(g2, w3[...], preferred_element_type=jnp.float32) + b3[...]
        m = jnp.mean(g3, axis=-1, keepdims=True)
        v = jnp.mean((g3 - m) ** 2, axis=-1, keepdims=True)
        g = (g3 - m) * lax.rsqrt(v + 1e-5) * lg[...] + lb[...]
        xn = x[...] + g
        out_x[...] = xn
        if emit_next:
            out_pd[...] = jnp.dot(xn, wd[...],
                                  preferred_element_type=jnp.float32)
            out_ps[...] = jnp.dot(xn, ws[...],
                                  preferred_element_type=jnp.float32)

    nbs = pl.BlockSpec((BN, D), lambda i: (i, 0))
    a0bs = pl.BlockSpec((1, BN, D), lambda i: (0, i, 0))
    a1bs = pl.BlockSpec((1, BN, D), lambda i: (1, i, 0))
    wbs = pl.BlockSpec((D, H), lambda i: (0, 0))
    bbs = pl.BlockSpec((1, H), lambda i: (0, 0))
    in_specs = [nbs, a0bs, a1bs, wbs, wbs, wbs, wbs, bbs, bbs, bbs, bbs, bbs]
    out_specs = [nbs]
    out_shape = [jax.ShapeDtypeStruct((N, D), jnp.float32)]
    if emit_next:
        in_specs += [wbs, wbs]
        out_specs += [nbs, nbs]
        out_shape += [jax.ShapeDtypeStruct((N, D), jnp.float32),
                      jax.ShapeDtypeStruct((N, D), jnp.float32)]
    return pl.pallas_call(
        body, grid=grid, in_specs=in_specs, out_specs=out_specs,
        out_shape=out_shape)


def _proj(N, D, H, BN):
    grid = (N // BN,)

    def body(x, wd, ws, out_pd, out_ps):
        out_pd[...] = jnp.dot(x[...], wd[...],
                              preferred_element_type=jnp.float32)
        out_ps[...] = jnp.dot(x[...], ws[...],
                              preferred_element_type=jnp.float32)

    nbs = pl.BlockSpec((BN, D), lambda i: (i, 0))
    wbs = pl.BlockSpec((D, H), lambda i: (0, 0))
    return pl.pallas_call(
        body, grid=grid, in_specs=[nbs, wbs, wbs], out_specs=[nbs, nbs],
        out_shape=[jax.ShapeDtypeStruct((N, H), jnp.float32),
                   jax.ShapeDtypeStruct((N, H), jnp.float32)])


def kernel(x, edge_index, edge_attr, We1, be1, We2, be2, We3, be3, lne_g,
           lne_b, Wn1, bn1, Wn2, bn2, Wn3, bn3, lnn_g, lnn_b):
    N, D = x.shape
    E = edge_index.shape[1]
    H = We2.shape[1]
    steps = We1.shape[0]
    src = edge_index[0]
    dst = edge_index[1]

    BE = 2000
    BN = 2000

    gather_k = _gather_combine(E, N, D)
    scatter_k = _scatter_partial(E, D)
    edge_k = _edge_mlp(E, D, H, BE)
    node_next_k = _node_mlp(N, D, H, BN, True)
    node_last_k = _node_mlp(N, D, H, BN, False)
    proj_k = _proj(N, D, H, BN)

    pd, ps = proj_k(x, We1[0, :D], We1[0, D:2 * D])
    for st in range(steps):
        g = gather_k(pd, ps, dst, src)
        edge_attr, h = edge_k(
            edge_attr, g, We1[st, 2 * D:], We2[st], We3[st],
            be1[st].reshape(1, H), be2[st].reshape(1, H),
            be3[st].reshape(1, D), lne_g[st].reshape(1, D),
            lne_b[st].reshape(1, D))
        part = scatter_k(h, dst)
        common = (x, part, part, Wn1[st, :D], Wn1[st, D:], Wn2[st], Wn3[st],
                  bn1[st].reshape(1, H), bn2[st].reshape(1, H),
                  bn3[st].reshape(1, D), lnn_g[st].reshape(1, D),
                  lnn_b[st].reshape(1, D))
        if st + 1 < steps:
            x, pd, ps = node_next_k(*common, We1[st + 1, :D],
                                    We1[st + 1, D:2 * D])
        else:
            (x,) = node_last_k(*common)
    return x


# speedup vs baseline: 4.6874x; 1.6452x over previous
import functools

import jax
import jax.numpy as jnp
from jax import lax
from jax.experimental import pallas as pl
from jax.experimental.pallas import tpu as pltpu
from jax.experimental.pallas import tpu_sc as plsc

NC = 2
NS = 16
NW = NC * NS

C = 80
ROWS_PER_TILE = 640
N_PAD = NS * ROWS_PER_TILE


def _gather_combine(E, N, D):
    per_w = E // NW
    n_chunks = per_w // C
    assert per_w % C == 0 and E % NW == 0
    n_main = n_chunks - 1 if n_chunks % 2 else n_chunks - 2
    mesh = plsc.VectorSubcoreMesh(core_axis_name="c", subcore_axis_name="s")

    @functools.partial(
        pl.kernel,
        mesh=mesh,
        out_type=jax.ShapeDtypeStruct((E, D), jnp.float32),
        scratch_types=[
            pltpu.VMEM((1, n_chunks, C), jnp.int32),
            pltpu.VMEM((1, n_chunks, C), jnp.int32),
            pltpu.VMEM((2, C, D), jnp.float32),
            pltpu.VMEM((2, C, D), jnp.float32),
            pltpu.VMEM((2, C, D), jnp.float32),
            pltpu.SemaphoreType.DMA,
            pltpu.SemaphoreType.DMA,
            pltpu.SemaphoreType.DMA,
            pltpu.SemaphoreType.DMA,
            pltpu.SemaphoreType.DMA,
            pltpu.SemaphoreType.DMA,
        ],
    )
    def k(pd_hbm, ps_hbm, dst_hbm, src_hbm, out_hbm, idxd, idxs, bufa, bufb,
          bufo, sa0, sa1, sb0, sb1, so0, so1):
        wid = lax.axis_index("s") * NC + lax.axis_index("c")
        base = wid * per_w
        pltpu.sync_copy(dst_hbm.at[pl.ds(wid, 1)], idxd)
        pltpu.sync_copy(src_hbm.at[pl.ds(wid, 1)], idxs)
        sa = (sa0, sa1)
        sb = (sb0, sb1)
        so = (so0, so1)

        def issue(i, b):
            pltpu.async_copy(pd_hbm.at[idxd.at[0, i]], bufa.at[b], sa[b])
            pltpu.async_copy(ps_hbm.at[idxs.at[0, i]], bufb.at[b], sb[b])

        def wait_gather(i, b):
            pltpu.make_async_copy(pd_hbm.at[idxd.at[0, i]], bufa.at[b],
                                  sa[b]).wait()
            pltpu.make_async_copy(ps_hbm.at[idxs.at[0, i]], bufb.at[b],
                                  sb[b]).wait()

        def add_rows(b):
            def row(r, carry):
                for j in range(D // 16):
                    sl = pl.ds(j * 16, 16)
                    bufo[b, r, sl] = bufa[b, r, sl] + bufb[b, r, sl]
                return carry

            lax.fori_loop(0, C, row, 0)

        def wait_store(i, b):
            pltpu.make_async_copy(
                bufo.at[b], out_hbm.at[pl.ds(base + i * C, C)],
                so[b]).wait()

        def do_chunk(i, b, last):
            @pl.when(i + 1 < n_chunks)
            def _():
                issue(i + 1, 1 - b)

            wait_gather(i, b)

            @pl.when(i >= 2)
            def _():
                wait_store(i - 2, b)

            add_rows(b)
            pltpu.async_copy(bufo.at[b],
                             out_hbm.at[pl.ds(base + i * C, C)], so[b])

        issue(0, 0)

        def pair(j, carry):
            do_chunk(2 * j, 0, False)
            do_chunk(2 * j + 1, 1, False)
            return carry

        lax.fori_loop(0, n_main // 2, pair, 0)
        for t in range(n_main, n_chunks):
            do_chunk(t, t % 2, t == n_chunks - 1)
        wait_store(n_chunks - 2, (n_chunks - 2) % 2)
        wait_store(n_chunks - 1, (n_chunks - 1) % 2)

    return k


def _scatter_partial(E, D):
    per_t = E // NW
    n_chunks = per_t // C
    nz_chunks = ROWS_PER_TILE // C
    mesh = plsc.VectorSubcoreMesh(core_axis_name="c", subcore_axis_name="s")

    n_main = n_chunks - 1 if n_chunks % 2 else n_chunks - 2

    @functools.partial(
        pl.kernel,
        mesh=mesh,
        out_type=jax.ShapeDtypeStruct((NC, N_PAD, D), jnp.float32),
        scratch_types=[
            pltpu.VMEM((1, n_chunks, C), jnp.int32),
            pltpu.VMEM((2, C, D), jnp.float32),
            pltpu.VMEM((C, D), jnp.float32),
            pltpu.VMEM_SHARED((N_PAD, D), jnp.float32),
            pltpu.SemaphoreType.DMA,
            pltpu.SemaphoreType.DMA,
            pltpu.SemaphoreType.DMA,
            pltpu.SemaphoreType.DMA,
        ],
    )
    def k(h_hbm, dst_hbm, out_hbm, idxd, bufh, zbuf, acc, sh0, sh1, ss0, ss1):
        c = lax.axis_index("c")
        s = lax.axis_index("s")
        wid = s * NC + c
        pltpu.sync_copy(dst_hbm.at[pl.ds(wid, 1)], idxd)

        def zrow(r, carry):
            for j in range(D // 16):
                zbuf[r, pl.ds(j * 16, 16)] = jnp.zeros((16,), jnp.float32)
            return carry

        lax.fori_loop(0, C, zrow, 0)
        row0 = s * ROWS_PER_TILE

        def zchunk(i, carry):
            pltpu.sync_copy(zbuf, acc.at[pl.ds(row0 + i * C, C)])
            return carry

        lax.fori_loop(0, nz_chunks, zchunk, 0)
        plsc.subcore_barrier()

        base = wid * per_t
        sh = (sh0, sh1)
        ss = (ss0, ss1)

        def issue_load(i, b):
            pltpu.async_copy(h_hbm.at[pl.ds(base + i * C, C)], bufh.at[b],
                             sh[b])

        def wait_load(i, b):
            pltpu.make_async_copy(h_hbm.at[pl.ds(base + i * C, C)],
                                  bufh.at[b], sh[b]).wait()

        def wait_scatter(i, b):
            pltpu.make_async_copy(bufh.at[b], acc.at[idxd.at[0, i]],
                                  ss[b]).wait()

        def do_chunk(i, b):
            @pl.when(i + 1 < n_chunks)
            def _():
                @pl.when(i >= 1)
                def _():
                    wait_scatter(i - 1, 1 - b)

                issue_load(i + 1, 1 - b)

            wait_load(i, b)
            pltpu.async_copy(bufh.at[b], acc.at[idxd.at[0, i]], ss[b],
                             add=True)

        issue_load(0, 0)

        def pair(j, carry):
            do_chunk(2 * j, 0)
            do_chunk(2 * j + 1, 1)
            return carry

        lax.fori_loop(0, n_main // 2, pair, 0)
        for t in range(n_main, n_chunks):
            do_chunk(t, t % 2)
        wait_scatter(n_chunks - 2, (n_chunks - 2) % 2)
        wait_scatter(n_chunks - 1, (n_chunks - 1) % 2)
        plsc.subcore_barrier()

        def ochunk(i, carry):
            r0 = row0 + i * C
            pltpu.sync_copy(acc.at[pl.ds(r0, C)], out_hbm.at[c, pl.ds(r0, C)])
            return carry

        lax.fori_loop(0, nz_chunks, ochunk, 0)

    return k


def _edge_mlp(E, D, H, BE):
    grid = (E // BE,)

    def body(ea, g, wa, w2, w3, b1, b2, b3, lg, lb, out_ea, out_h):
        h1 = jnp.maximum(
            jnp.dot(ea[...], wa[...], preferred_element_type=jnp.float32)
            + g[...] + b1[...], 0.0)
        h2 = jnp.maximum(
            jnp.dot(h1, w2[...], preferred_element_type=jnp.float32)
            + b2[...], 0.0)
        h3 = jnp.dot(h2, w3[...], preferred_element_type=jnp.float32) + b3[...]
        m = jnp.mean(h3, axis=-1, keepdims=True)
        v = jnp.mean((h3 - m) ** 2, axis=-1, keepdims=True)
        h = (h3 - m) * lax.rsqrt(v + 1e-5) * lg[...] + lb[...]
        out_ea[...] = ea[...] + h
        out_h[...] = h

    ebs = pl.BlockSpec((BE, D), lambda i: (i, 0))
    wbs = pl.BlockSpec((D, H), lambda i: (0, 0))
    bbs = pl.BlockSpec((1, H), lambda i: (0, 0))
    return pl.pallas_call(
        body,
        grid=grid,
        in_specs=[ebs, ebs, wbs, wbs, wbs, bbs, bbs, bbs, bbs, bbs],
        out_specs=[ebs, ebs],
        out_shape=[
            jax.ShapeDtypeStruct((E, D), jnp.float32),
            jax.ShapeDtypeStruct((E, D), jnp.float32),
        ],
    )


def _node_mlp(N, D, H, BN, emit_next):
    grid = (N // BN,)

    def body(*refs):
        if emit_next:
            (x, a0, a1, w1x, w1a, w2, w3, b1, b2, b3, lg, lb, wd, ws,
             out_x, out_pd, out_ps) = refs
        else:
            (x, a0, a1, w1x, w1a, w2, w3, b1, b2, b3, lg, lb,
             out_x) = refs
        aggr = a0[0] + a1[0]
        g1 = jnp.maximum(
            jnp.dot(x[...], w1x[...], preferred_element_type=jnp.float32)
            + jnp.dot(aggr, w1a[...], preferred_element_type=jnp.float32)
            + b1[...], 0.0)
        g2 = jnp.maximum(
            jnp.dot(g1, w2[...], preferred_element_type=jnp.float32)
            + b2[...], 0.0)
        g3 = jnp.dot(g2, w3[...], preferred_element_type=jnp.float32) + b3[...]
        m = jnp.mean(g3, axis=-1, keepdims=True)
        v = jnp.mean((g3 - m) ** 2, axis=-1, keepdims=True)
        g = (g3 - m) * lax.rsqrt(v + 1e-5) * lg[...] + lb[...]
        xn = x[...] + g
        out_x[...] = xn
        if emit_next:
            out_pd[...] = jnp.dot(xn, wd[...],
                                  preferred_element_type=jnp.float32)
            out_ps[...] = jnp.dot(xn, ws[...],
                                  preferred_element_type=jnp.float32)

    nbs = pl.BlockSpec((BN, D), lambda i: (i, 0))
    a0bs = pl.BlockSpec((1, BN, D), lambda i: (0, i, 0))
    a1bs = pl.BlockSpec((1, BN, D), lambda i: (1, i, 0))
    wbs = pl.BlockSpec((D, H), lambda i: (0, 0))
    bbs = pl.BlockSpec((1, H), lambda i: (0, 0))
    in_specs = [nbs, a0bs, a1bs, wbs, wbs, wbs, wbs, bbs, bbs, bbs, bbs, bbs]
    out_specs = [nbs]
    out_shape = [jax.ShapeDtypeStruct((N, D), jnp.float32)]
    if emit_next:
        in_specs += [wbs, wbs]
        out_specs += [nbs, nbs]
        out_shape += [jax.ShapeDtypeStruct((N, D), jnp.float32),
                      jax.ShapeDtypeStruct((N, D), jnp.float32)]
    return pl.pallas_call(
        body, grid=grid, in_specs=in_specs, out_specs=out_specs,
        out_shape=out_shape)


def _proj(N, D, H, BN):
    grid = (N // BN,)

    def body(x, wd, ws, out_pd, out_ps):
        out_pd[...] = jnp.dot(x[...], wd[...],
                              preferred_element_type=jnp.float32)
        out_ps[...] = jnp.dot(x[...], ws[...],
                              preferred_element_type=jnp.float32)

    nbs = pl.BlockSpec((BN, D), lambda i: (i, 0))
    wbs = pl.BlockSpec((D, H), lambda i: (0, 0))
    return pl.pallas_call(
        body, grid=grid, in_specs=[nbs, wbs, wbs], out_specs=[nbs, nbs],
        out_shape=[jax.ShapeDtypeStruct((N, H), jnp.float32),
                   jax.ShapeDtypeStruct((N, H), jnp.float32)])


def kernel(x, edge_index, edge_attr, We1, be1, We2, be2, We3, be3, lne_g,
           lne_b, Wn1, bn1, Wn2, bn2, Wn3, bn3, lnn_g, lnn_b):
    N, D = x.shape
    E = edge_index.shape[1]
    H = We2.shape[1]
    steps = We1.shape[0]
    src = edge_index[0]
    dst = edge_index[1]
    n_chunks = E // NW // C
    dst_w = dst.reshape(NW, n_chunks, C)
    src_w = src.reshape(NW, n_chunks, C)

    BE = 2000
    BN = 2000

    gather_k = _gather_combine(E, N, D)
    scatter_k = _scatter_partial(E, D)
    edge_k = _edge_mlp(E, D, H, BE)
    node_next_k = _node_mlp(N, D, H, BN, True)
    node_last_k = _node_mlp(N, D, H, BN, False)
    proj_k = _proj(N, D, H, BN)

    pd, ps = proj_k(x, We1[0, :D], We1[0, D:2 * D])
    for st in range(steps):
        g = gather_k(pd, ps, dst_w, src_w)
        edge_attr, h = edge_k(
            edge_attr, g, We1[st, 2 * D:], We2[st], We3[st],
            be1[st].reshape(1, H), be2[st].reshape(1, H),
            be3[st].reshape(1, D), lne_g[st].reshape(1, D),
            lne_b[st].reshape(1, D))
        part = scatter_k(h, dst_w)
        common = (x, part, part, Wn1[st, :D], Wn1[st, D:], Wn2[st], Wn3[st],
                  bn1[st].reshape(1, H), bn2[st].reshape(1, H),
                  bn3[st].reshape(1, D), lnn_g[st].reshape(1, D),
                  lnn_b[st].reshape(1, D))
        if st + 1 < steps:
            x, pd, ps = node_next_k(*common, We1[st + 1, :D],
                                    We1[st + 1, D:2 * D])
        else:
            (x,) = node_last_k(*common)
    return x


# speedup vs baseline: 4.8554x; 1.0358x over previous
import functools

import jax
import jax.numpy as jnp
from jax import lax
from jax.experimental import pallas as pl
from jax.experimental.pallas import tpu as pltpu
from jax.experimental.pallas import tpu_sc as plsc

NC = 2
NS = 16
NW = NC * NS

C = 80
ROWS_PER_TILE = 640
N_PAD = NS * ROWS_PER_TILE


def _gather_combine(E, N, D):
    per_w = E // NW
    n_chunks = per_w // C
    assert per_w % C == 0 and E % NW == 0
    n_main = n_chunks - 1 if n_chunks % 2 else n_chunks - 2
    mesh = plsc.VectorSubcoreMesh(core_axis_name="c", subcore_axis_name="s")

    @functools.partial(
        pl.kernel,
        mesh=mesh,
        out_type=jax.ShapeDtypeStruct((E, D), jnp.float32),
        scratch_types=[
            pltpu.VMEM((1, n_chunks, C), jnp.int32),
            pltpu.VMEM((1, n_chunks, C), jnp.int32),
            pltpu.VMEM((2, C, D), jnp.float32),
            pltpu.VMEM((2, C, D), jnp.float32),
            pltpu.VMEM((2, C, D), jnp.float32),
            pltpu.SemaphoreType.DMA,
            pltpu.SemaphoreType.DMA,
            pltpu.SemaphoreType.DMA,
            pltpu.SemaphoreType.DMA,
            pltpu.SemaphoreType.DMA,
            pltpu.SemaphoreType.DMA,
        ],
    )
    def k(pd_hbm, ps_hbm, dst_hbm, src_hbm, out_hbm, idxd, idxs, bufa, bufb,
          bufo, sa0, sa1, sb0, sb1, so0, so1):
        wid = lax.axis_index("s") * NC + lax.axis_index("c")
        base = wid * per_w
        pltpu.sync_copy(dst_hbm.at[pl.ds(wid, 1)], idxd)
        pltpu.sync_copy(src_hbm.at[pl.ds(wid, 1)], idxs)
        sa = (sa0, sa1)
        sb = (sb0, sb1)
        so = (so0, so1)

        def issue(i, b):
            pltpu.async_copy(pd_hbm.at[idxd.at[0, i]], bufa.at[b], sa[b])
            pltpu.async_copy(ps_hbm.at[idxs.at[0, i]], bufb.at[b], sb[b])

        def wait_gather(i, b):
            pltpu.make_async_copy(pd_hbm.at[idxd.at[0, i]], bufa.at[b],
                                  sa[b]).wait()
            pltpu.make_async_copy(ps_hbm.at[idxs.at[0, i]], bufb.at[b],
                                  sb[b]).wait()

        def add_rows(b):
            def row(r, carry):
                for j in range(D // 16):
                    sl = pl.ds(j * 16, 16)
                    bufo[b, r, sl] = bufa[b, r, sl] + bufb[b, r, sl]
                return carry

            lax.fori_loop(0, C, row, 0)

        def wait_store(i, b):
            pltpu.make_async_copy(
                bufo.at[b], out_hbm.at[pl.ds(base + i * C, C)],
                so[b]).wait()

        def do_chunk(i, b):
            @pl.when(i + 1 < n_chunks)
            def _():
                issue(i + 1, 1 - b)

            wait_gather(i, b)

            @pl.when(i >= 2)
            def _():
                wait_store(i - 2, b)

            add_rows(b)
            pltpu.async_copy(bufo.at[b],
                             out_hbm.at[pl.ds(base + i * C, C)], so[b])

        issue(0, 0)

        def pair(j, carry):
            do_chunk(2 * j, 0)
            do_chunk(2 * j + 1, 1)
            return carry

        lax.fori_loop(0, n_main // 2, pair, 0)
        for t in range(n_main, n_chunks):
            do_chunk(t, t % 2)
        wait_store(n_chunks - 2, (n_chunks - 2) % 2)
        wait_store(n_chunks - 1, (n_chunks - 1) % 2)

    return k


def _scatter_partial(E, D):
    per_t = E // NW
    n_chunks = per_t // C
    nz_chunks = ROWS_PER_TILE // C
    mesh = plsc.VectorSubcoreMesh(core_axis_name="c", subcore_axis_name="s")

    n_main = n_chunks - 1 if n_chunks % 2 else n_chunks - 2

    @functools.partial(
        pl.kernel,
        mesh=mesh,
        out_type=jax.ShapeDtypeStruct((NC, N_PAD, D), jnp.float32),
        scratch_types=[
            pltpu.VMEM((1, n_chunks, C), jnp.int32),
            pltpu.VMEM((2, C, D), jnp.float32),
            pltpu.VMEM((C, D), jnp.float32),
            pltpu.VMEM_SHARED((N_PAD, D), jnp.float32),
            pltpu.SemaphoreType.DMA,
            pltpu.SemaphoreType.DMA,
            pltpu.SemaphoreType.DMA,
            pltpu.SemaphoreType.DMA,
        ],
    )
    def k(h_hbm, dst_hbm, out_hbm, idxd, bufh, zbuf, acc, sh0, sh1, ss0, ss1):
        c = lax.axis_index("c")
        s = lax.axis_index("s")
        wid = s * NC + c
        pltpu.sync_copy(dst_hbm.at[pl.ds(wid, 1)], idxd)

        def zrow(r, carry):
            for j in range(D // 16):
                zbuf[r, pl.ds(j * 16, 16)] = jnp.zeros((16,), jnp.float32)
            return carry

        lax.fori_loop(0, C, zrow, 0)
        row0 = s * ROWS_PER_TILE

        def zchunk(i, carry):
            pltpu.sync_copy(zbuf, acc.at[pl.ds(row0 + i * C, C)])
            return carry

        lax.fori_loop(0, nz_chunks, zchunk, 0)
        plsc.subcore_barrier()

        base = wid * per_t
        sh = (sh0, sh1)
        ss = (ss0, ss1)

        def issue_load(i, b):
            pltpu.async_copy(h_hbm.at[pl.ds(base + i * C, C)], bufh.at[b],
                             sh[b])

        def wait_load(i, b):
            pltpu.make_async_copy(h_hbm.at[pl.ds(base + i * C, C)],
                                  bufh.at[b], sh[b]).wait()

        def wait_scatter(i, b):
            pltpu.make_async_copy(bufh.at[b], acc.at[idxd.at[0, i]],
                                  ss[b]).wait()

        def do_chunk(i, b):
            @pl.when(i + 1 < n_chunks)
            def _():
                @pl.when(i >= 1)
                def _():
                    wait_scatter(i - 1, 1 - b)

                issue_load(i + 1, 1 - b)

            wait_load(i, b)
            pltpu.async_copy(bufh.at[b], acc.at[idxd.at[0, i]], ss[b],
                             add=True)

        issue_load(0, 0)

        def pair(j, carry):
            do_chunk(2 * j, 0)
            do_chunk(2 * j + 1, 1)
            return carry

        lax.fori_loop(0, n_main // 2, pair, 0)
        for t in range(n_main, n_chunks):
            do_chunk(t, t % 2)
        wait_scatter(n_chunks - 2, (n_chunks - 2) % 2)
        wait_scatter(n_chunks - 1, (n_chunks - 1) % 2)
        plsc.subcore_barrier()

        def ochunk(i, carry):
            r0 = row0 + i * C
            pltpu.sync_copy(acc.at[pl.ds(r0, C)], out_hbm.at[c, pl.ds(r0, C)])
            return carry

        lax.fori_loop(0, nz_chunks, ochunk, 0)

    return k


def _edge_mlp(E, D, H, BE, ea_dtype):
    grid = (E // BE,)

    def body(ea, g, wa, w2, w3, b1, b2, b3, lg, lb, out_ea, out_h):
        eaf = ea[...].astype(jnp.float32)
        h1 = jnp.maximum(
            jnp.dot(eaf, wa[...], preferred_element_type=jnp.float32)
            + g[...] + b1[...], 0.0)
        h2 = jnp.maximum(
            jnp.dot(h1, w2[...], preferred_element_type=jnp.float32)
            + b2[...], 0.0)
        h3 = jnp.dot(h2, w3[...], preferred_element_type=jnp.float32) + b3[...]
        m = jnp.mean(h3, axis=-1, keepdims=True)
        v = jnp.mean((h3 - m) ** 2, axis=-1, keepdims=True)
        h = (h3 - m) * lax.rsqrt(v + 1e-5) * lg[...] + lb[...]
        out_ea[...] = (eaf + h).astype(jnp.bfloat16)
        out_h[...] = h

    ebs = pl.BlockSpec((BE, D), lambda i: (i, 0))
    wbs = pl.BlockSpec((D, H), lambda i: (0, 0))
    bbs = pl.BlockSpec((1, H), lambda i: (0, 0))
    return pl.pallas_call(
        body,
        grid=grid,
        in_specs=[ebs, ebs, wbs, wbs, wbs, bbs, bbs, bbs, bbs, bbs],
        out_specs=[ebs, ebs],
        out_shape=[
            jax.ShapeDtypeStruct((E, D), jnp.bfloat16),
            jax.ShapeDtypeStruct((E, D), jnp.float32),
        ],
    )


def _node_mlp(N, D, H, BN, emit_next):
    grid = (N // BN,)

    def body(*refs):
        if emit_next:
            (x, a0, a1, w1x, w1a, w2, w3, b1, b2, b3, lg, lb, wd, ws,
             out_x, out_pd, out_ps) = refs
        else:
            (x, a0, a1, w1x, w1a, w2, w3, b1, b2, b3, lg, lb,
             out_x) = refs
        aggr = a0[0] + a1[0]
        g1 = jnp.maximum(
            jnp.dot(x[...], w1x[...], preferred_element_type=jnp.float32)
            + jnp.dot(aggr, w1a[...], preferred_element_type=jnp.float32)
            + b1[...], 0.0)
        g2 = jnp.maximum(
            jnp.dot(g1, w2[...], preferred_element_type=jnp.float32)
            + b2[...], 0.0)
        g3 = jnp.dot(g2, w3[...], preferred_element_type=jnp.float32) + b3[...]
        m = jnp.mean(g3, axis=-1, keepdims=True)
        v = jnp.mean((g3 - m) ** 2, axis=-1, keepdims=True)
        g = (g3 - m) * lax.rsqrt(v + 1e-5) * lg[...] + lb[...]
        xn = x[...] + g
        out_x[...] = xn
        if emit_next:
            out_pd[...] = jnp.dot(xn, wd[...],
                                  preferred_element_type=jnp.float32)
            out_ps[...] = jnp.dot(xn, ws[...],
                                  preferred_element_type=jnp.float32)

    nbs = pl.BlockSpec((BN, D), lambda i: (i, 0))
    a0bs = pl.BlockSpec((1, BN, D), lambda i: (0, i, 0))
    a1bs = pl.BlockSpec((1, BN, D), lambda i: (1, i, 0))
    wbs = pl.BlockSpec((D, H), lambda i: (0, 0))
    bbs = pl.BlockSpec((1, H), lambda i: (0, 0))
    in_specs = [nbs, a0bs, a1bs, wbs, wbs, wbs, wbs, bbs, bbs, bbs, bbs, bbs]
    out_specs = [nbs]
    out_shape = [jax.ShapeDtypeStruct((N, D), jnp.float32)]
    if emit_next:
        in_specs += [wbs, wbs]
        out_specs += [nbs, nbs]
        out_shape += [jax.ShapeDtypeStruct((N, D), jnp.float32),
                      jax.ShapeDtypeStruct((N, D), jnp.float32)]
    return pl.pallas_call(
        body, grid=grid, in_specs=in_specs, out_specs=out_specs,
        out_shape=out_shape)


def _proj(N, D, H, BN):
    grid = (N // BN,)

    def body(x, wd, ws, out_pd, out_ps):
        out_pd[...] = jnp.dot(x[...], wd[...],
                              preferred_element_type=jnp.float32)
        out_ps[...] = jnp.dot(x[...], ws[...],
                              preferred_element_type=jnp.float32)

    nbs = pl.BlockSpec((BN, D), lambda i: (i, 0))
    wbs = pl.BlockSpec((D, H), lambda i: (0, 0))
    return pl.pallas_call(
        body, grid=grid, in_specs=[nbs, wbs, wbs], out_specs=[nbs, nbs],
        out_shape=[jax.ShapeDtypeStruct((N, H), jnp.float32),
                   jax.ShapeDtypeStruct((N, H), jnp.float32)])


def kernel(x, edge_index, edge_attr, We1, be1, We2, be2, We3, be3, lne_g,
           lne_b, Wn1, bn1, Wn2, bn2, Wn3, bn3, lnn_g, lnn_b):
    N, D = x.shape
    E = edge_index.shape[1]
    H = We2.shape[1]
    steps = We1.shape[0]
    src = edge_index[0]
    dst = edge_index[1]
    n_chunks = E // NW // C
    dst_w = dst.reshape(NW, n_chunks, C)
    src_w = src.reshape(NW, n_chunks, C)

    BE = 2000
    BN = 2000

    gather_k = _gather_combine(E, N, D)
    scatter_k = _scatter_partial(E, D)
    edge_k0 = _edge_mlp(E, D, H, BE, jnp.float32)
    edge_k1 = _edge_mlp(E, D, H, BE, jnp.bfloat16)
    node_next_k = _node_mlp(N, D, H, BN, True)
    node_last_k = _node_mlp(N, D, H, BN, False)
    proj_k = _proj(N, D, H, BN)

    pd, ps = proj_k(x, We1[0, :D], We1[0, D:2 * D])
    for st in range(steps):
        g = gather_k(pd, ps, dst_w, src_w)
        edge_attr, h = (edge_k0 if st == 0 else edge_k1)(
            edge_attr, g, We1[st, 2 * D:], We2[st], We3[st],
            be1[st].reshape(1, H), be2[st].reshape(1, H),
            be3[st].reshape(1, D), lne_g[st].reshape(1, D),
            lne_b[st].reshape(1, D))
        part = scatter_k(h, dst_w)
        common = (x, part, part, Wn1[st, :D], Wn1[st, D:], Wn2[st], Wn3[st],
                  bn1[st].reshape(1, H), bn2[st].reshape(1, H),
                  bn3[st].reshape(1, D), lnn_g[st].reshape(1, D),
                  lnn_b[st].reshape(1, D))
        if st + 1 < steps:
            x, pd, ps = node_next_k(*common, We1[st + 1, :D],
                                    We1[st + 1, D:2 * D])
        else:
            (x,) = node_last_k(*common)
    return x


# speedup vs baseline: 5.2913x; 1.0898x over previous
import functools

import jax
import jax.numpy as jnp
from jax import lax
from jax.experimental import pallas as pl
from jax.experimental.pallas import tpu as pltpu
from jax.experimental.pallas import tpu_sc as plsc

NC = 2
NS = 16
NW = NC * NS

C = 80
ROWS_PER_TILE = 640
N_PAD = NS * ROWS_PER_TILE


def _gather_combine(E, N, D, Ck):
    per_w = E // NW
    n_chunks = per_w // Ck
    assert per_w % Ck == 0 and E % NW == 0
    n_main = n_chunks - 1 if n_chunks % 2 else n_chunks - 2
    mesh = plsc.VectorSubcoreMesh(core_axis_name="c", subcore_axis_name="s")

    @functools.partial(
        pl.kernel,
        mesh=mesh,
        out_type=jax.ShapeDtypeStruct((E, D), jnp.float32),
        scratch_types=[
            pltpu.VMEM((1, n_chunks, Ck), jnp.int32),
            pltpu.VMEM((1, n_chunks, Ck), jnp.int32),
            pltpu.VMEM((2, Ck, D), jnp.float32),
            pltpu.VMEM((2, Ck, D), jnp.float32),
            pltpu.VMEM((2, Ck, D), jnp.float32),
            pltpu.SemaphoreType.DMA,
            pltpu.SemaphoreType.DMA,
            pltpu.SemaphoreType.DMA,
            pltpu.SemaphoreType.DMA,
            pltpu.SemaphoreType.DMA,
            pltpu.SemaphoreType.DMA,
        ],
    )
    def k(pd_hbm, ps_hbm, dst_hbm, src_hbm, out_hbm, idxd, idxs, bufa, bufb,
          bufo, sa0, sa1, sb0, sb1, so0, so1):
        wid = lax.axis_index("s") * NC + lax.axis_index("c")
        base = wid * per_w
        pltpu.sync_copy(dst_hbm.at[pl.ds(wid, 1)], idxd)
        pltpu.sync_copy(src_hbm.at[pl.ds(wid, 1)], idxs)
        sa = (sa0, sa1)
        sb = (sb0, sb1)
        so = (so0, so1)

        def issue(i, b):
            pltpu.async_copy(pd_hbm.at[idxd.at[0, i]], bufa.at[b], sa[b])
            pltpu.async_copy(ps_hbm.at[idxs.at[0, i]], bufb.at[b], sb[b])

        def wait_gather(i, b):
            pltpu.make_async_copy(pd_hbm.at[idxd.at[0, i]], bufa.at[b],
                                  sa[b]).wait()
            pltpu.make_async_copy(ps_hbm.at[idxs.at[0, i]], bufb.at[b],
                                  sb[b]).wait()

        def add_rows(b):
            def rowpair(r2, carry):
                for rr in range(2):
                    r = r2 * 2 + rr
                    for j in range(D // 16):
                        sl = pl.ds(j * 16, 16)
                        bufo[b, r, sl] = bufa[b, r, sl] + bufb[b, r, sl]
                return carry

            lax.fori_loop(0, Ck // 2, rowpair, 0)

        def wait_store(i, b):
            pltpu.make_async_copy(
                bufo.at[b], out_hbm.at[pl.ds(base + i * Ck, Ck)],
                so[b]).wait()

        def do_chunk(i, b):
            @pl.when(i + 1 < n_chunks)
            def _():
                issue(i + 1, 1 - b)

            wait_gather(i, b)

            @pl.when(i >= 2)
            def _():
                wait_store(i - 2, b)

            add_rows(b)
            pltpu.async_copy(bufo.at[b],
                             out_hbm.at[pl.ds(base + i * Ck, Ck)], so[b])

        issue(0, 0)

        def pair(j, carry):
            do_chunk(2 * j, 0)
            do_chunk(2 * j + 1, 1)
            return carry

        lax.fori_loop(0, n_main // 2, pair, 0)
        for t in range(n_main, n_chunks):
            do_chunk(t, t % 2)
        wait_store(n_chunks - 2, (n_chunks - 2) % 2)
        wait_store(n_chunks - 1, (n_chunks - 1) % 2)

    return k


def _scatter_partial(E, D, Ck):
    per_t = E // NW
    n_chunks = per_t // Ck
    nz_chunks = ROWS_PER_TILE // Ck
    mesh = plsc.VectorSubcoreMesh(core_axis_name="c", subcore_axis_name="s")

    n_main = n_chunks - 1 if n_chunks % 2 else n_chunks - 2

    @functools.partial(
        pl.kernel,
        mesh=mesh,
        out_type=jax.ShapeDtypeStruct((NC, N_PAD, D), jnp.float32),
        scratch_types=[
            pltpu.VMEM((1, n_chunks, Ck), jnp.int32),
            pltpu.VMEM((2, Ck, D), jnp.float32),
            pltpu.VMEM((Ck, D), jnp.float32),
            pltpu.VMEM_SHARED((N_PAD, D), jnp.float32),
            pltpu.SemaphoreType.DMA,
            pltpu.SemaphoreType.DMA,
            pltpu.SemaphoreType.DMA,
            pltpu.SemaphoreType.DMA,
        ],
    )
    def k(h_hbm, dst_hbm, out_hbm, idxd, bufh, zbuf, acc, sh0, sh1, ss0, ss1):
        c = lax.axis_index("c")
        s = lax.axis_index("s")
        wid = s * NC + c
        base = wid * per_t
        sh = (sh0, sh1)
        ss = (ss0, ss1)
        pltpu.async_copy(h_hbm.at[pl.ds(base, Ck)], bufh.at[0], sh[0])
        pltpu.sync_copy(dst_hbm.at[pl.ds(wid, 1)], idxd)

        def zrow(r, carry):
            for j in range(D // 16):
                zbuf[r, pl.ds(j * 16, 16)] = jnp.zeros((16,), jnp.float32)
            return carry

        lax.fori_loop(0, Ck, zrow, 0)
        row0 = s * ROWS_PER_TILE

        def zchunk(i, carry):
            pltpu.sync_copy(zbuf, acc.at[pl.ds(row0 + i * Ck, Ck)])
            return carry

        lax.fori_loop(0, nz_chunks, zchunk, 0)
        plsc.subcore_barrier()

        def issue_load(i, b):
            pltpu.async_copy(h_hbm.at[pl.ds(base + i * Ck, Ck)], bufh.at[b],
                             sh[b])

        def wait_load(i, b):
            pltpu.make_async_copy(h_hbm.at[pl.ds(base + i * Ck, Ck)],
                                  bufh.at[b], sh[b]).wait()

        def wait_scatter(i, b):
            pltpu.make_async_copy(bufh.at[b], acc.at[idxd.at[0, i]],
                                  ss[b]).wait()

        def do_chunk(i, b):
            @pl.when(i + 1 < n_chunks)
            def _():
                @pl.when(i >= 1)
                def _():
                    wait_scatter(i - 1, 1 - b)

                issue_load(i + 1, 1 - b)

            wait_load(i, b)
            pltpu.async_copy(bufh.at[b], acc.at[idxd.at[0, i]], ss[b],
                             add=True)

        def pair(j, carry):
            do_chunk(2 * j, 0)
            do_chunk(2 * j + 1, 1)
            return carry

        lax.fori_loop(0, n_main // 2, pair, 0)
        for t in range(n_main, n_chunks):
            do_chunk(t, t % 2)
        wait_scatter(n_chunks - 2, (n_chunks - 2) % 2)
        wait_scatter(n_chunks - 1, (n_chunks - 1) % 2)
        plsc.subcore_barrier()

        def ochunk(i, carry):
            r0 = row0 + i * Ck
            pltpu.sync_copy(acc.at[pl.ds(r0, Ck)],
                            out_hbm.at[c, pl.ds(r0, Ck)])
            return carry

        lax.fori_loop(0, nz_chunks, ochunk, 0)

    return k


def _edge_mlp(E2, D, H, BE, variant, half=0):
    grid = (E2 // BE,)

    def body(*refs):
        bf = jnp.bfloat16
        if variant == "first":
            ea, g, wa, w2, w3, b1, b2, b3, lg, lb, out_ea, out_h = refs
            eab = ea[0].astype(bf)
            eaf = ea[0].astype(jnp.float32)
        else:
            ea, g, wa, w2, w3, b1, b2, b3, lg, lb, out_h = refs
            eab = ea[...]
            eaf = ea[...].astype(jnp.float32)
        h1 = jnp.maximum(
            jnp.dot(eab, wa[...].astype(bf), preferred_element_type=jnp.float32)
            + g[...] + b1[...], 0.0)
        h2 = jnp.maximum(
            jnp.dot(h1.astype(bf), w2[...].astype(bf),
                    preferred_element_type=jnp.float32) + b2[...], 0.0)
        h3 = jnp.dot(h2.astype(bf), w3[...].astype(bf),
                     preferred_element_type=jnp.float32) + b3[...]
        m = jnp.mean(h3, axis=-1, keepdims=True)
        v = jnp.mean((h3 - m) ** 2, axis=-1, keepdims=True)
        h = (h3 - m) * lax.rsqrt(v + 1e-5) * lg[...] + lb[...]
        if variant == "first":
            out_ea[...] = (eaf + h).astype(jnp.bfloat16)
        out_h[...] = h

    ebs = pl.BlockSpec((BE, D), lambda i: (i, 0))
    fbs = pl.BlockSpec((1, BE, D), lambda i: (half, i, 0))
    wbs = pl.BlockSpec((D, H), lambda i: (0, 0))
    bbs = pl.BlockSpec((1, H), lambda i: (0, 0))
    if variant == "first":
        in_specs = [fbs, ebs, wbs, wbs, wbs, bbs, bbs, bbs, bbs, bbs]
        out_specs = [ebs, ebs]
        out_shape = [jax.ShapeDtypeStruct((E2, D), jnp.bfloat16),
                     jax.ShapeDtypeStruct((E2, D), jnp.float32)]
    else:
        in_specs = [ebs, ebs, wbs, wbs, wbs, bbs, bbs, bbs, bbs, bbs]
        out_specs = [ebs]
        out_shape = [jax.ShapeDtypeStruct((E2, D), jnp.float32)]
    return pl.pallas_call(
        body, grid=grid, in_specs=in_specs, out_specs=out_specs,
        out_shape=out_shape)


def _node_mlp(N, D, H, BN, emit_next):
    grid = (N // BN,)

    def body(*refs):
        if emit_next:
            (x, a00, a01, a10, a11, w1x, w1a, w2, w3, b1, b2, b3, lg, lb,
             wd, ws, out_x, out_pd, out_ps) = refs
        else:
            (x, a00, a01, a10, a11, w1x, w1a, w2, w3, b1, b2, b3, lg, lb,
             out_x) = refs
        aggr = (a00[0] + a01[0]) + (a10[0] + a11[0])
        g1 = jnp.maximum(
            jnp.dot(x[...], w1x[...], preferred_element_type=jnp.float32)
            + jnp.dot(aggr, w1a[...], preferred_element_type=jnp.float32)
            + b1[...], 0.0)
        g2 = jnp.maximum(
            jnp.dot(g1, w2[...], preferred_element_type=jnp.float32)
            + b2[...], 0.0)
        g3 = jnp.dot(g2, w3[...], preferred_element_type=jnp.float32) + b3[...]
        m = jnp.mean(g3, axis=-1, keepdims=True)
        v = jnp.mean((g3 - m) ** 2, axis=-1, keepdims=True)
        g = (g3 - m) * lax.rsqrt(v + 1e-5) * lg[...] + lb[...]
        xn = x[...] + g
        out_x[...] = xn
        if emit_next:
            out_pd[...] = jnp.dot(xn, wd[...],
                                  preferred_element_type=jnp.float32)
            out_ps[...] = jnp.dot(xn, ws[...],
                                  preferred_element_type=jnp.float32)

    nbs = pl.BlockSpec((BN, D), lambda i: (i, 0))
    a0bs = pl.BlockSpec((1, BN, D), lambda i: (0, i, 0))
    a1bs = pl.BlockSpec((1, BN, D), lambda i: (1, i, 0))
    wbs = pl.BlockSpec((D, H), lambda i: (0, 0))
    bbs = pl.BlockSpec((1, H), lambda i: (0, 0))
    in_specs = [nbs, a0bs, a1bs, a0bs, a1bs,
                wbs, wbs, wbs, wbs, bbs, bbs, bbs, bbs, bbs]
    out_specs = [nbs]
    out_shape = [jax.ShapeDtypeStruct((N, D), jnp.float32)]
    if emit_next:
        in_specs += [wbs, wbs]
        out_specs += [nbs, nbs]
        out_shape += [jax.ShapeDtypeStruct((N, D), jnp.float32),
                      jax.ShapeDtypeStruct((N, D), jnp.float32)]
    return pl.pallas_call(
        body, grid=grid, in_specs=in_specs, out_specs=out_specs,
        out_shape=out_shape)


def _proj(N, D, H, BN):
    grid = (N // BN,)

    def body(x, wd, ws, out_pd, out_ps):
        out_pd[...] = jnp.dot(x[...], wd[...],
                              preferred_element_type=jnp.float32)
        out_ps[...] = jnp.dot(x[...], ws[...],
                              preferred_element_type=jnp.float32)

    nbs = pl.BlockSpec((BN, D), lambda i: (i, 0))
    wbs = pl.BlockSpec((D, H), lambda i: (0, 0))
    return pl.pallas_call(
        body, grid=grid, in_specs=[nbs, wbs, wbs], out_specs=[nbs, nbs],
        out_shape=[jax.ShapeDtypeStruct((N, H), jnp.float32),
                   jax.ShapeDtypeStruct((N, H), jnp.float32)])


def kernel(x, edge_index, edge_attr, We1, be1, We2, be2, We3, be3, lne_g,
           lne_b, Wn1, bn1, Wn2, bn2, Wn3, bn3, lnn_g, lnn_b):
    N, D = x.shape
    E = edge_index.shape[1]
    H = We2.shape[1]
    steps = We1.shape[0]
    src = edge_index[0]
    dst = edge_index[1]

    E2 = E // 2
    CH = 40
    nch = E2 // NW // CH
    dst_w = [dst[:E2].reshape(NW, nch, CH), dst[E2:].reshape(NW, nch, CH)]
    src_w = [src[:E2].reshape(NW, nch, CH), src[E2:].reshape(NW, nch, CH)]

    BE = 4000
    BN = 2000

    gather_k = _gather_combine(E2, N, D, CH)
    scatter_k = _scatter_partial(E2, D, CH)
    edge_first = [_edge_mlp(E2, D, H, BE, "first", half=hh)
                  for hh in range(2)]
    edge_last = _edge_mlp(E2, D, H, BE, "last")
    node_next_k = _node_mlp(N, D, H, BN, True)
    node_last_k = _node_mlp(N, D, H, BN, False)
    proj_k = _proj(N, D, H, BN)

    ea_full = edge_attr.reshape(2, E2, D)
    ea_half = [None, None]
    pd, ps = proj_k(x, We1[0, :D], We1[0, D:2 * D])
    for st in range(steps):
        ew = (We1[st, 2 * D:], We2[st], We3[st],
              be1[st].reshape(1, H), be2[st].reshape(1, H),
              be3[st].reshape(1, D), lne_g[st].reshape(1, D),
              lne_b[st].reshape(1, D))
        parts = [None, None]
        for hh in range(2):
            g = gather_k(pd, ps, dst_w[hh], src_w[hh])
            if st == 0:
                ea_half[hh], hm = edge_first[hh](ea_full, g, *ew)
            else:
                (hm,) = edge_last(ea_half[hh], g, *ew)
            parts[hh] = scatter_k(hm, dst_w[hh])
        common = (x, parts[0], parts[0], parts[1], parts[1],
                  Wn1[st, :D], Wn1[st, D:], Wn2[st], Wn3[st],
                  bn1[st].reshape(1, H), bn2[st].reshape(1, H),
                  bn3[st].reshape(1, D), lnn_g[st].reshape(1, D),
                  lnn_b[st].reshape(1, D))
        if st + 1 < steps:
            x, pd, ps = node_next_k(*common, We1[st + 1, :D],
                                    We1[st + 1, D:2 * D])
        else:
            (x,) = node_last_k(*common)
    return x
